# Initial kernel scaffold; baseline (speedup 1.0000x reference)
import jax
import jax.numpy as jnp
from jax.experimental import pallas as pl


def kernel(x, W_p, b_p):
    raise NotImplementedError("write your pallas kernel here")


# speedup vs baseline: 3.3305x; 3.3305x over previous
import functools

import jax
import jax.numpy as jnp
from jax import lax
from jax.experimental import pallas as pl
from jax.experimental.pallas import tpu as pltpu
from jax.experimental.pallas import tpu_sc as plsc

_TAPS = [(ic, dz, dy, dx) for dz in range(3) for dy in range(3)
         for dx in range(3) for ic in range(4)]
_KP = 112
_MP = 9472
_LW = 9728
_TBL = 941192
_CH = 4736
_NV = _CH // 16
_NR = _CH // 128


def _tc_body(x0r, x1r, x2r, wr, br, q_ref, at_ref):
    z = pl.program_id(0)
    xr = [x0r, x1r, x2r]
    for k, (ic, dz, dy, dx) in enumerate(_TAPS):
        off = dy * 98 + dx
        at_ref[k, :] = xr[dz][0, ic, off:off + _MP]
    r = jnp.dot(wr[...], at_ref[...], preferred_element_type=jnp.float32)
    t = r + br[...][:, :1]
    iota_m = lax.broadcasted_iota(jnp.int32, (16, _MP), 1)
    gy = (iota_m // 98 + 1).astype(jnp.float32)
    gx = (iota_m % 98 + 1).astype(jnp.float32)
    zf = (z + 1).astype(jnp.float32)
    rowi = lax.broadcasted_iota(jnp.int32, (16, _MP), 0)
    grid = jnp.where(rowi < 3, zf, jnp.where(rowi < 6, gy, gx))
    q = grid + t
    q = jnp.minimum(jnp.maximum(q, 0.0), 96.0)
    q_ref[0] = q


def _tc_stage(xpf, wm, bias):
    return pl.pallas_call(
        _tc_body,
        grid=(96,),
        in_specs=[
            pl.BlockSpec((1, 4, _LW), lambda z: (z, 0, 0)),
            pl.BlockSpec((1, 4, _LW), lambda z: (z + 1, 0, 0)),
            pl.BlockSpec((1, 4, _LW), lambda z: (z + 2, 0, 0)),
            pl.BlockSpec((16, _KP), lambda z: (0, 0)),
            pl.BlockSpec((16, 128), lambda z: (0, 0)),
        ],
        out_specs=pl.BlockSpec((1, 16, _MP), lambda z: (z, 0, 0)),
        out_shape=jax.ShapeDtypeStruct((96, 16, _MP), jnp.float32),
        scratch_shapes=[pltpu.VMEM((_KP, _MP), jnp.float32)],
    )(xpf, xpf, xpf, wm, bias)


def _sc_body(q_hbm, tab_hbm, out_hbm, qzv, qyv, qxv, fzv, fyv, fxv, idxv,
             gv, outv, sem):
    info = plsc.get_sparse_core_info()
    nc = info.num_cores
    wid = lax.axis_index("s") * nc + lax.axis_index("c")
    qrefs = (qzv, qyv, qxv)

    def unit_body(u, carry):
        uid = wid * 9 + u
        z = uid // 3
        ch = uid - z * 3
        chbase = ch * _TBL
        for h in range(2):
            base = h * _CH
            for c in range(3):
                pltpu.sync_copy(q_hbm.at[z, c * 3 + ch, pl.ds(base, _CH)],
                                qrefs[c])

            def idx_body(i, cc):
                sl = pl.ds(i * 16, 16)
                qz = qzv[sl]
                qy = qyv[sl]
                qx = qxv[sl]
                az0 = qz * 9604.0
                az1 = (qz + 1.0) * 9604.0
                by0 = qy * 98.0
                by1 = (qy + 1.0) * 98.0
                corner = 0
                for az in (az0, az1):
                    for by in (by0, by1):
                        for bx in (qx, qx + 1.0):
                            s = (az + by) + bx
                            iv = s.astype(jnp.int32) + chbase
                            idxv[pl.ds(corner * _CH + i * 16, 16)] = iv
                            corner += 1
                fzv[sl] = qz.astype(jnp.int32).astype(jnp.float32) - qz
                fyv[sl] = qy.astype(jnp.int32).astype(jnp.float32) - qy
                fxv[sl] = qx.astype(jnp.int32).astype(jnp.float32) - qx
                return cc

            lax.fori_loop(0, _NV, idx_body, 0)

            for c8 in range(8):
                def fire_body(r2, cc, c8=c8):
                    pltpu.async_copy(
                        tab_hbm.at[idxv.at[pl.ds(c8 * _CH + r2 * 128, 128)]],
                        gv[c8].at[pl.ds(r2 * 128, 128)], sem)
                    return cc
                lax.fori_loop(0, _NR, fire_body, 0)
            for c8 in range(8):
                def drain_body(r2, cc, c8=c8):
                    pltpu.make_async_copy(
                        tab_hbm.at[idxv.at[pl.ds(c8 * _CH + r2 * 128, 128)]],
                        gv[c8].at[pl.ds(r2 * 128, 128)], sem).wait()
                    return cc
                lax.fori_loop(0, _NR, drain_body, 0)

            def comb_body(i, cc):
                sl = pl.ds(i * 16, 16)
                fz = fzv[sl]
                fy = fyv[sl]
                fx = fxv[sl]
                u0 = 1.0 + fz
                u1 = -fz
                v0 = 1.0 + fy
                v1 = -fy
                t0 = 1.0 + fx
                t1 = -fx
                g = [gv[c][sl] for c in range(8)]
                rz0 = v0 * (t0 * g[0] + t1 * g[1]) + v1 * (t0 * g[2] + t1 * g[3])
                rz1 = v0 * (t0 * g[4] + t1 * g[5]) + v1 * (t0 * g[6] + t1 * g[7])
                outv[sl] = u0 * rz0 + u1 * rz1
                return cc

            lax.fori_loop(0, _NV, comb_body, 0)
            pltpu.sync_copy(outv, out_hbm.at[ch, z, pl.ds(base, _CH)])
        return carry

    lax.fori_loop(0, 9, unit_body, 0)


@functools.partial(
    pl.kernel,
    out_type=jax.ShapeDtypeStruct((3, 96, _MP), jnp.float32),
    mesh=plsc.VectorSubcoreMesh(core_axis_name="c", subcore_axis_name="s"),
    scratch_types=(
        [pltpu.VMEM((_CH,), jnp.float32)] * 6
        + [pltpu.VMEM((8 * _CH,), jnp.int32)]
        + [pltpu.VMEM((_CH,), jnp.float32)] * 8
        + [pltpu.VMEM((_CH,), jnp.float32), pltpu.SemaphoreType.DMA]
    ),
)
def _sc_stage(q_hbm, tab_hbm, out_hbm, qzv, qyv, qxv, fzv, fyv, fxv, idxv,
              g0, g1, g2, g3, g4, g5, g6, g7, outv, sem):
    _sc_body(q_hbm, tab_hbm, out_hbm, qzv, qyv, qxv, fzv, fyv, fxv, idxv,
             (g0, g1, g2, g3, g4, g5, g6, g7), outv, sem)


def kernel(x, W_p, b_p):
    xpf = jnp.pad(x[0], ((0, 0), (1, 1), (1, 1), (1, 1))).reshape(4, 98, 9604)
    xpf = jnp.pad(xpf, ((0, 0), (0, 0), (0, _LW - 9604))).transpose(1, 0, 2)
    wm = jnp.stack([W_p[:, ic, dz, dy, dx] for (ic, dz, dy, dx) in _TAPS], axis=1)
    wm = jnp.pad(wm, ((0, 16 - 9), (0, _KP - 108)))
    bias = jnp.broadcast_to(jnp.pad(b_p, (0, 16 - 9))[:, None], (16, 128))

    q = _tc_stage(xpf, wm, bias)

    tab = jnp.pad(x[0, :3], ((0, 0), (1, 1), (1, 1), (1, 1))).reshape(3 * _TBL)
    out = _sc_stage(q, tab)

    outm = out[:, :, :9408].reshape(3, 96, 96, 98)[..., :96][None]
    return jnp.concatenate([outm, x[:, 3:4]], axis=1)


# speedup vs baseline: 3.4652x; 1.0404x over previous
import functools

import jax
import jax.numpy as jnp
from jax import lax
from jax.experimental import pallas as pl
from jax.experimental.pallas import tpu as pltpu
from jax.experimental.pallas import tpu_sc as plsc

_TAPS = [(ic, dz, dy, dx) for dz in range(3) for dy in range(3)
         for dx in range(3) for ic in range(4)]
_KP = 112
_MP = 9472
_LW = 9728
_TBL = 941192
_CH = 4736
_NV = _CH // 16
_NR = _CH // 128


def _tc_body(x0r, x1r, x2r, wr, br, q_ref, at_ref):
    z = pl.program_id(0)
    xr = [x0r, x1r, x2r]
    for k, (ic, dz, dy, dx) in enumerate(_TAPS):
        off = dy * 98 + dx
        at_ref[k, :] = xr[dz][0, ic, off:off + _MP]
    r = jnp.dot(wr[...], at_ref[...], preferred_element_type=jnp.float32)
    t = r + br[...][:, :1]
    iota_m = lax.broadcasted_iota(jnp.int32, (16, _MP), 1)
    gy = (iota_m // 98 + 1).astype(jnp.float32)
    gx = (iota_m % 98 + 1).astype(jnp.float32)
    zf = (z + 1).astype(jnp.float32)
    rowi = lax.broadcasted_iota(jnp.int32, (16, _MP), 0)
    grid = jnp.where(rowi < 3, zf, jnp.where(rowi < 6, gy, gx))
    q = grid + t
    q = jnp.minimum(jnp.maximum(q, 0.0), 96.0)
    q_ref[0] = q


def _tc_stage(xpf, wm, bias):
    return pl.pallas_call(
        _tc_body,
        grid=(96,),
        in_specs=[
            pl.BlockSpec((1, 4, _LW), lambda z: (z, 0, 0)),
            pl.BlockSpec((1, 4, _LW), lambda z: (z + 1, 0, 0)),
            pl.BlockSpec((1, 4, _LW), lambda z: (z + 2, 0, 0)),
            pl.BlockSpec((16, _KP), lambda z: (0, 0)),
            pl.BlockSpec((16, 128), lambda z: (0, 0)),
        ],
        out_specs=pl.BlockSpec((1, 16, _MP), lambda z: (z, 0, 0)),
        out_shape=jax.ShapeDtypeStruct((96, 16, _MP), jnp.float32),
        scratch_shapes=[pltpu.VMEM((_KP, _MP), jnp.float32)],
    )(xpf, xpf, xpf, wm, bias)


def _sc_body(q_hbm, tab_hbm, out_hbm, qzv, qyv, qxv, fzv, fyv, fxv, iv8,
             gv, outv, sem):
    info = plsc.get_sparse_core_info()
    nc = info.num_cores
    wid = lax.axis_index("s") * nc + lax.axis_index("c")
    qrefs = (qzv, qyv, qxv)

    def unit_body(u, carry):
        uid = wid * 9 + u
        z = uid // 3
        ch = uid - z * 3
        chbase = ch * _TBL
        for h in range(2):
            base = h * _CH
            for c in range(3):
                pltpu.sync_copy(q_hbm.at[z, c * 3 + ch, pl.ds(base, _CH)],
                                qrefs[c])

            def idx_body(i, cc):
                sl = pl.ds(i * 16, 16)
                qz = qzv[sl]
                qy = qyv[sl]
                qx = qxv[sl]
                az0 = qz * 9604.0
                az1 = (qz + 1.0) * 9604.0
                by0 = qy * 98.0
                by1 = (qy + 1.0) * 98.0
                corner = 0
                for az in (az0, az1):
                    for by in (by0, by1):
                        for bx in (qx, qx + 1.0):
                            s = (az + by) + bx
                            iv = s.astype(jnp.int32) + chbase
                            iv8[corner][sl] = iv
                            corner += 1
                fzv[sl] = qz.astype(jnp.int32).astype(jnp.float32) - qz
                fyv[sl] = qy.astype(jnp.int32).astype(jnp.float32) - qy
                fxv[sl] = qx.astype(jnp.int32).astype(jnp.float32) - qx
                return cc

            lax.fori_loop(0, _NV, idx_body, 0)

            for c8 in range(8):
                pltpu.async_copy(tab_hbm.at[iv8[c8]], gv[c8], sem)
            for c8 in range(8):
                pltpu.make_async_copy(tab_hbm.at[iv8[c8]], gv[c8], sem).wait()

            def comb_body(i, cc):
                sl = pl.ds(i * 16, 16)
                fz = fzv[sl]
                fy = fyv[sl]
                fx = fxv[sl]
                u0 = 1.0 + fz
                u1 = -fz
                v0 = 1.0 + fy
                v1 = -fy
                t0 = 1.0 + fx
                t1 = -fx
                g = [gv[c][sl] for c in range(8)]
                rz0 = v0 * (t0 * g[0] + t1 * g[1]) + v1 * (t0 * g[2] + t1 * g[3])
                rz1 = v0 * (t0 * g[4] + t1 * g[5]) + v1 * (t0 * g[6] + t1 * g[7])
                outv[sl] = u0 * rz0 + u1 * rz1
                return cc

            lax.fori_loop(0, _NV, comb_body, 0)
            pltpu.sync_copy(outv, out_hbm.at[ch, z, pl.ds(base, _CH)])
        return carry

    lax.fori_loop(0, 9, unit_body, 0)


@functools.partial(
    pl.kernel,
    out_type=jax.ShapeDtypeStruct((3, 96, _MP), jnp.float32),
    mesh=plsc.VectorSubcoreMesh(core_axis_name="c", subcore_axis_name="s"),
    scratch_types=(
        [pltpu.VMEM((_CH,), jnp.float32)] * 6
        + [pltpu.VMEM((_CH,), jnp.int32)] * 8
        + [pltpu.VMEM((_CH,), jnp.float32)] * 8
        + [pltpu.VMEM((_CH,), jnp.float32), pltpu.SemaphoreType.DMA]
    ),
)
def _sc_stage(q_hbm, tab_hbm, out_hbm, qzv, qyv, qxv, fzv, fyv, fxv,
              i0, i1, i2, i3, i4, i5, i6, i7,
              g0, g1, g2, g3, g4, g5, g6, g7, outv, sem):
    _sc_body(q_hbm, tab_hbm, out_hbm, qzv, qyv, qxv, fzv, fyv, fxv,
             (i0, i1, i2, i3, i4, i5, i6, i7),
             (g0, g1, g2, g3, g4, g5, g6, g7), outv, sem)


def kernel(x, W_p, b_p):
    xpf = jnp.pad(x[0], ((0, 0), (1, 1), (1, 1), (1, 1))).reshape(4, 98, 9604)
    xpf = jnp.pad(xpf, ((0, 0), (0, 0), (0, _LW - 9604))).transpose(1, 0, 2)
    wm = jnp.stack([W_p[:, ic, dz, dy, dx] for (ic, dz, dy, dx) in _TAPS], axis=1)
    wm = jnp.pad(wm, ((0, 16 - 9), (0, _KP - 108)))
    bias = jnp.broadcast_to(jnp.pad(b_p, (0, 16 - 9))[:, None], (16, 128))

    q = _tc_stage(xpf, wm, bias)

    tab = jnp.pad(x[0, :3], ((0, 0), (1, 1), (1, 1), (1, 1))).reshape(3 * _TBL)
    out = _sc_stage(q, tab)

    outm = out[:, :, :9408].reshape(3, 96, 96, 98)[..., :96][None]
    return jnp.concatenate([outm, x[:, 3:4]], axis=1)
